# baseline (speedup 1.0000x reference)
import functools

import jax
import jax.numpy as jnp
from jax.experimental import pallas as pl
from jax.experimental.pallas import tpu as pltpu

_S = 4096
_K = 12
_CB = 512


def _topk_weights(p, iota):
    def body(_, carry):
        pv, accw = carry
        mx = jnp.max(pv, axis=1, keepdims=True)
        eq = pv == mx
        first = jnp.min(jnp.where(eq, iota, _S), axis=1, keepdims=True)
        sel = iota == first
        accw = accw + sel.astype(jnp.float32)
        pv = jnp.where(sel, -jnp.inf, pv)
        return pv, accw

    _, accw = jax.lax.fori_loop(
        0, _K, body, (p, jnp.zeros_like(p)), unroll=True
    )
    return accw


def _weights_kernel(out_ref, tau_ref, w_ref):
    o0 = out_ref[:, 0, :]
    o1 = out_ref[:, 1, :]
    m = jnp.maximum(o0, o1)
    e0 = jnp.exp(o0 - m)
    e1 = jnp.exp(o1 - m)
    s = e0 + e1
    pf = e1 / s
    pb = e0 / s

    tau = tau_ref[0, 0]
    fg_thres = jax.nn.sigmoid(tau)
    bg_thres = 1.0 - fg_thres

    iota = jax.lax.broadcasted_iota(jnp.int32, pf.shape, 1)
    inv_k = jnp.float32(1.0 / _K)

    for cls, p, thres in ((0, pf, fg_thres), (1, pb, bg_thres)):
        mask = (p > thres).astype(jnp.float32)
        cnt = jnp.sum(mask, axis=1, keepdims=True)
        tk = _topk_weights(p, iota)
        safe = jnp.where(cnt > 0, cnt, jnp.float32(1.0))
        w = jnp.where(cnt > 0, mask / safe, tk * inv_k)
        w_ref[:, cls, :] = w


def _dot_ws(w, feat):
    return jax.lax.dot_general(
        w, feat, (((1,), (1,)), ((), ())),
        preferred_element_type=jnp.float32,
    )


def _pool_kernel(fa_ref, fb_ref, fc_ref, fd_ref, w_ref, out_ref):
    w0 = w_ref[0]
    w1 = w_ref[1]
    out_ref[0] = jnp.concatenate(
        [_dot_ws(w0, fa_ref[0, 0]), _dot_ws(w0, fb_ref[0, 0])], axis=1)
    out_ref[1] = jnp.concatenate(
        [_dot_ws(w1, fc_ref[0, 0]), _dot_ws(w1, fd_ref[0, 0])], axis=1)


@jax.jit
def _run(feature_q, out, tau):
    bs, C = feature_q.shape[0], feature_q.shape[1]
    feat = feature_q.reshape(bs, C, _S)
    logits = out.reshape(bs, 2, _S)
    tau_arr = jnp.reshape(tau.astype(jnp.float32), (1, 1))

    w = pl.pallas_call(
        _weights_kernel,
        in_specs=[
            pl.BlockSpec((bs, 2, _S), lambda: (0, 0, 0)),
            pl.BlockSpec((1, 1), lambda: (0, 0)),
        ],
        out_specs=pl.BlockSpec((bs, 2, _S), lambda: (0, 0, 0)),
        out_shape=jax.ShapeDtypeStruct((bs, 2, _S), jnp.float32),
    )(logits, tau_arr)

    half = C // 2
    featv = feature_q.reshape(bs, 2, half, _S)
    protos = pl.pallas_call(
        _pool_kernel,
        grid=(bs // 2,),
        compiler_params=pltpu.CompilerParams(
            dimension_semantics=(pltpu.PARALLEL,),
        ),
        in_specs=[
            pl.BlockSpec((1, 1, half, _S), lambda a: (2 * a, 0, 0, 0)),
            pl.BlockSpec((1, 1, half, _S), lambda a: (2 * a, 1, 0, 0)),
            pl.BlockSpec((1, 1, half, _S), lambda a: (2 * a + 1, 0, 0, 0)),
            pl.BlockSpec((1, 1, half, _S), lambda a: (2 * a + 1, 1, 0, 0)),
            pl.BlockSpec((2, 2, _S), lambda a: (a, 0, 0)),
        ],
        out_specs=pl.BlockSpec((2, 2, C), lambda a: (a, 0, 0)),
        out_shape=jax.ShapeDtypeStruct((bs, 2, C), jnp.float32),
    )(featv, featv, featv, featv, w)

    fg = protos[:, 0, :].reshape(bs, C, 1, 1)
    bg = protos[:, 1, :].reshape(bs, C, 1, 1)
    return fg, bg


def kernel(feature_q, out, tau):
    return _run(feature_q, out, jnp.asarray(tau))


# speedup vs baseline: 2.3489x; 2.3489x over previous
import jax
import jax.numpy as jnp
from jax.experimental import pallas as pl
from jax.experimental.pallas import tpu as pltpu

_S = 4096
_K = 12


def _topk_weights(p, iota):
    def body(_, carry):
        pv, accw = carry
        mx = jnp.max(pv, axis=1, keepdims=True)
        eq = pv == mx
        first = jnp.min(jnp.where(eq, iota, _S), axis=1, keepdims=True)
        sel = iota == first
        accw = accw + sel.astype(jnp.float32)
        pv = jnp.where(sel, -jnp.inf, pv)
        return pv, accw

    _, accw = jax.lax.fori_loop(
        0, _K, body, (p, jnp.zeros_like(p)), unroll=True
    )
    return accw


def _fused_kernel(logits_ref, tau_ref, feat_ref, out_ref, w_scratch):
    i = pl.program_id(0)

    @pl.when(i == 0)
    def _weights():
        o0 = logits_ref[:, 0, :]
        o1 = logits_ref[:, 1, :]
        m = jnp.maximum(o0, o1)
        e0 = jnp.exp(o0 - m)
        e1 = jnp.exp(o1 - m)
        s = e0 + e1
        pf = e1 / s
        pb = e0 / s

        tau = tau_ref[0, 0]
        fg_thres = jax.nn.sigmoid(tau)
        bg_thres = 1.0 - fg_thres

        iota = jax.lax.broadcasted_iota(jnp.int32, pf.shape, 1)
        inv_k = jnp.float32(1.0 / _K)

        for cls, p, thres in ((0, pf, fg_thres), (1, pb, bg_thres)):
            mask = (p > thres).astype(jnp.float32)
            cnt = jnp.sum(mask, axis=1, keepdims=True)
            tk = _topk_weights(p, iota)
            safe = jnp.where(cnt > 0, cnt, jnp.float32(1.0))
            w = jnp.where(cnt > 0, mask / safe, tk * inv_k)
            w_scratch[:, cls, :] = w

    @pl.when(i > 0)
    def _pool():
        b = i - 1
        w = w_scratch[pl.ds(b, 1), :, :][0]
        out_ref[0] = jax.lax.dot_general(
            w, feat_ref[0], (((1,), (1,)), ((), ())),
            preferred_element_type=jnp.float32,
        )


@jax.jit
def _run(feature_q, out, tau):
    bs, C = feature_q.shape[0], feature_q.shape[1]
    feat = feature_q.reshape(bs, C, _S)
    logits = out.reshape(bs, 2, _S)
    tau_arr = jnp.reshape(tau.astype(jnp.float32), (1, 1))

    protos = pl.pallas_call(
        _fused_kernel,
        grid=(bs + 1,),
        in_specs=[
            pl.BlockSpec((bs, 2, _S), lambda i: (0, 0, 0)),
            pl.BlockSpec((1, 1), lambda i: (0, 0)),
            pl.BlockSpec((1, C, _S), lambda i: (jnp.maximum(i - 1, 0), 0, 0)),
        ],
        out_specs=pl.BlockSpec((1, 2, C), lambda i: (jnp.maximum(i - 1, 0), 0, 0)),
        out_shape=jax.ShapeDtypeStruct((bs, 2, C), jnp.float32),
        scratch_shapes=[pltpu.VMEM((bs, 2, _S), jnp.float32)],
    )(logits, tau_arr, feat)

    fg = protos[:, 0, :].reshape(bs, C, 1, 1)
    bg = protos[:, 1, :].reshape(bs, C, 1, 1)
    return fg, bg


def kernel(feature_q, out, tau):
    return _run(feature_q, out, jnp.asarray(tau))


# speedup vs baseline: 2.4194x; 1.0300x over previous
import jax
import jax.numpy as jnp
from jax.experimental import pallas as pl
from jax.experimental.pallas import tpu as pltpu

_S = 4096
_K = 12
_CB = 256
_NBUF = 4


def _topk_weights(p, iota):
    def body(_, carry):
        pv, accw = carry
        mx = jnp.max(pv, axis=1, keepdims=True)
        eq = pv == mx
        first = jnp.min(jnp.where(eq, iota, _S), axis=1, keepdims=True)
        sel = iota == first
        accw = accw + sel.astype(jnp.float32)
        pv = jnp.where(sel, -jnp.inf, pv)
        return pv, accw

    _, accw = jax.lax.fori_loop(
        0, _K, body, (p, jnp.zeros_like(p)), unroll=True
    )
    return accw


def _make_kernel(bs, C):
    nc = C // _CB
    tot = bs * nc

    def fused(logits_ref, tau_ref, feat_hbm, out_ref, fbuf, sems):
        def start_copy(c, slot):
            b, jc = divmod(c, nc)
            pltpu.make_async_copy(
                feat_hbm.at[b, pl.ds(jc * _CB, _CB), :],
                fbuf.at[slot],
                sems.at[slot],
            ).start()

        for c in range(min(_NBUF, tot)):
            start_copy(c, c)

        o0 = logits_ref[:, 0, :]
        o1 = logits_ref[:, 1, :]
        m = jnp.maximum(o0, o1)
        e0 = jnp.exp(o0 - m)
        e1 = jnp.exp(o1 - m)
        s = e0 + e1
        pf = e1 / s
        pb = e0 / s

        tau = tau_ref[0, 0]
        fg_thres = jax.nn.sigmoid(tau)
        bg_thres = 1.0 - fg_thres

        iota = jax.lax.broadcasted_iota(jnp.int32, pf.shape, 1)
        inv_k = jnp.float32(1.0 / _K)

        wvecs = []
        for p, thres in ((pf, fg_thres), (pb, bg_thres)):
            mask = (p > thres).astype(jnp.float32)
            cnt = jnp.sum(mask, axis=1, keepdims=True)
            tk = _topk_weights(p, iota)
            safe = jnp.where(cnt > 0, cnt, jnp.float32(1.0))
            wvecs.append(jnp.where(cnt > 0, mask / safe, tk * inv_k))
        wf, wb = wvecs

        for c in range(tot):
            slot = c % _NBUF
            b, jc = divmod(c, nc)
            pltpu.make_async_copy(
                feat_hbm.at[b, pl.ds(jc * _CB, _CB), :],
                fbuf.at[slot],
                sems.at[slot],
            ).wait()
            w = jnp.concatenate([wf[b:b + 1], wb[b:b + 1]], axis=0)
            res = jax.lax.dot_general(
                w, fbuf[slot], (((1,), (1,)), ((), ())),
                preferred_element_type=jnp.float32,
            )
            out_ref[b, :, pl.ds(jc * _CB, _CB)] = res
            if c + _NBUF < tot:
                start_copy(c + _NBUF, slot)

    return fused, tot


@jax.jit
def _run(feature_q, out, tau):
    bs, C = feature_q.shape[0], feature_q.shape[1]
    feat = feature_q.reshape(bs, C, _S)
    logits = out.reshape(bs, 2, _S)
    tau_arr = jnp.reshape(tau.astype(jnp.float32), (1, 1))

    fused, _ = _make_kernel(bs, C)
    protos = pl.pallas_call(
        fused,
        in_specs=[
            pl.BlockSpec((bs, 2, _S), lambda: (0, 0, 0)),
            pl.BlockSpec((1, 1), lambda: (0, 0)),
            pl.BlockSpec(memory_space=pltpu.HBM),
        ],
        out_specs=pl.BlockSpec((bs, 2, C), lambda: (0, 0, 0)),
        out_shape=jax.ShapeDtypeStruct((bs, 2, C), jnp.float32),
        scratch_shapes=[
            pltpu.VMEM((_NBUF, _CB, _S), jnp.float32),
            pltpu.SemaphoreType.DMA((_NBUF,)),
        ],
    )(logits, tau_arr, feat)

    fg = protos[:, 0, :].reshape(bs, C, 1, 1)
    bg = protos[:, 1, :].reshape(bs, C, 1, 1)
    return fg, bg


def kernel(feature_q, out, tau):
    return _run(feature_q, out, jnp.asarray(tau))
